# baseline (speedup 1.0000x reference)
import jax
import jax.numpy as jnp
from jax import lax
from jax.experimental import pallas as pl
from jax.experimental.pallas import tpu as pltpu
from jax.experimental.pallas import tpu_sc as plsc

_NUM_EMBEDDINGS = 1_000_000
_DIM = 64
_TP_SIZE = 4
_TP_RANK = 1
_PER_PART = _NUM_EMBEDDINGS // _TP_SIZE
_VSTART = _PER_PART * _TP_RANK
_VEND = _VSTART + _PER_PART

_ZPAD = 65536
_ZMASK = _ZPAD - 1

_B = 16384 * 50
_NC = 2
_NS = 16
_NW = _NC * _NS
_CHUNK = _B // _NW
_G = 128
_S = 512
_NGS = _S // _G
_NSUP = _CHUNK // _S
_NB = 3
_L = 16


def _body(x_hbm, tab_hbm, out_hbm, xin, lidx, rows, sem_i, sem_g, sem_w):
    wid = lax.axis_index("s") * _NC + lax.axis_index("c")
    base = wid * _CHUNK

    def fire_ids(sc, b):
        sbase = base + sc * _S
        pltpu.async_copy(x_hbm.at[pl.ds(sbase, _S)], xin.at[b], sem_i.at[b])

    def wait_ids(sc, b):
        sbase = base + sc * _S
        pltpu.make_async_copy(x_hbm.at[pl.ds(sbase, _S)], xin.at[b],
                              sem_i.at[b]).wait()

    def fire_gathers(sc, b):
        wait_ids(sc, b)
        for g in range(_NGS):
            def mk(i, c, g=g):
                xv = xin[b, pl.ds(g * _G + i * _L, _L)]
                m = (xv >= _VSTART) & (xv < _VEND)
                lidx[b, g, pl.ds(i * _L, _L)] = jnp.where(
                    m, xv - _VSTART, _PER_PART + (xv & _ZMASK))
                return c
            lax.fori_loop(0, _G // _L, mk, 0)
            pltpu.async_copy(tab_hbm.at[lidx.at[b, g]],
                             rows.at[b, pl.ds(g * _G, _G)], sem_g.at[b])

    def drain_gathers(b):
        for g in range(_NGS):
            pltpu.make_async_copy(tab_hbm.at[lidx.at[b, 0]],
                                  rows.at[b, pl.ds(0, _G)],
                                  sem_g.at[b]).wait()

    def writeback(sc, b):
        sbase = base + sc * _S
        pltpu.async_copy(rows.at[b], out_hbm.at[pl.ds(sbase, _S)], sem_w.at[b])

    def wait_wb(b):
        pltpu.make_async_copy(rows.at[b], out_hbm.at[pl.ds(base, _S)],
                              sem_w.at[b]).wait()

    fire_ids(0, 0)
    fire_ids(1, 1)
    fire_ids(2, 2)
    fire_gathers(0, 0)
    fire_gathers(1, 1)

    def step(sc, carry):
        b = sc % _NB
        drain_gathers(b)
        writeback(sc, b)
        bn = (sc + 2) % _NB
        @pl.when(sc + 2 < _NSUP)
        def _():
            @pl.when(sc > 0)
            def _():
                wait_wb(bn)
            fire_gathers(sc + 2, bn)

        @pl.when(sc + 3 < _NSUP)
        def _():
            fire_ids(sc + 3, (sc + 3) % _NB)
        return carry

    lax.fori_loop(0, _NSUP, step, 0)
    for b in range(_NB):
        wait_wb(b)


def kernel(x, embedding):
    xf = x.reshape(-1)
    tab = jnp.concatenate(
        [embedding, jnp.zeros((_ZPAD, _DIM), jnp.float32)], axis=0)
    mesh = plsc.VectorSubcoreMesh(core_axis_name="c", subcore_axis_name="s")
    f = pl.kernel(
        _body,
        out_type=jax.ShapeDtypeStruct((_B, _DIM), jnp.float32),
        mesh=mesh,
        compiler_params=pltpu.CompilerParams(use_tc_tiling_on_sc=False),
        scratch_types=[
            pltpu.VMEM((_NB, _S), jnp.int32),
            pltpu.VMEM((_NB, _NGS, _G), jnp.int32),
            pltpu.VMEM((_NB, _S, _DIM), jnp.float32),
            pltpu.SemaphoreType.DMA((_NB,)),
            pltpu.SemaphoreType.DMA((_NB,)),
            pltpu.SemaphoreType.DMA((_NB,)),
        ],
    )
    out = f(xf, tab)
    return out.reshape(x.shape[0], x.shape[1], _DIM)


# speedup vs baseline: 1.0288x; 1.0288x over previous
import jax
import jax.numpy as jnp
from jax import lax
from jax.experimental import pallas as pl
from jax.experimental.pallas import tpu as pltpu
from jax.experimental.pallas import tpu_sc as plsc

_NUM_EMBEDDINGS = 1_000_000
_DIM = 64
_TP_SIZE = 4
_TP_RANK = 1
_PER_PART = _NUM_EMBEDDINGS // _TP_SIZE
_VSTART = _PER_PART * _TP_RANK
_VEND = _VSTART + _PER_PART

_ZPAD = 8192
_ZMASK = _ZPAD - 1

_B = 16384 * 50
_NC = 2
_NS = 16
_NW = _NC * _NS
_CHUNK = _B // _NW
_G = 128
_S = 512
_NGS = _S // _G
_NSUP = _CHUNK // _S
_NB = 3
_L = 16


def _body(x_hbm, tab_hbm, out_hbm, xin, lidx, rows, sem_i, sem_g, sem_w):
    wid = lax.axis_index("s") * _NC + lax.axis_index("c")
    base = wid * _CHUNK

    def fire_ids(sc, b):
        sbase = base + sc * _S
        pltpu.async_copy(x_hbm.at[pl.ds(sbase, _S)], xin.at[b], sem_i.at[b])

    def wait_ids(sc, b):
        sbase = base + sc * _S
        pltpu.make_async_copy(x_hbm.at[pl.ds(sbase, _S)], xin.at[b],
                              sem_i.at[b]).wait()

    def fire_gathers(sc, b):
        wait_ids(sc, b)
        for g in range(_NGS):
            def mk(i, c, g=g):
                xv = xin[b, pl.ds(g * _G + i * _L, _L)]
                m = (xv >= _VSTART) & (xv < _VEND)
                lidx[b, g, pl.ds(i * _L, _L)] = jnp.where(
                    m, xv - _VSTART, _PER_PART + (xv & _ZMASK))
                return c
            lax.fori_loop(0, _G // _L, mk, 0)
            pltpu.async_copy(tab_hbm.at[lidx.at[b, g]],
                             rows.at[b, pl.ds(g * _G, _G)], sem_g.at[b])

    def drain_gathers(b):
        for g in range(_NGS):
            pltpu.make_async_copy(tab_hbm.at[lidx.at[b, 0]],
                                  rows.at[b, pl.ds(0, _G)],
                                  sem_g.at[b]).wait()

    def writeback(sc, b):
        sbase = base + sc * _S
        pltpu.async_copy(rows.at[b], out_hbm.at[pl.ds(sbase, _S)], sem_w.at[b])

    def wait_wb(b):
        pltpu.make_async_copy(rows.at[b], out_hbm.at[pl.ds(base, _S)],
                              sem_w.at[b]).wait()

    fire_ids(0, 0)
    fire_ids(1, 1)
    fire_ids(2, 2)
    fire_gathers(0, 0)
    fire_gathers(1, 1)

    def step(sc, carry):
        b = sc % _NB
        drain_gathers(b)
        writeback(sc, b)
        bn = (sc + 2) % _NB
        @pl.when(sc + 2 < _NSUP)
        def _():
            @pl.when(sc > 0)
            def _():
                wait_wb(bn)
            fire_gathers(sc + 2, bn)

        @pl.when(sc + 3 < _NSUP)
        def _():
            fire_ids(sc + 3, (sc + 3) % _NB)
        return carry

    lax.fori_loop(0, _NSUP, step, 0)
    for b in range(_NB):
        wait_wb(b)


def kernel(x, embedding):
    xf = x.reshape(-1)
    tab = jnp.concatenate(
        [embedding, jnp.zeros((_ZPAD, _DIM), jnp.float32)], axis=0)
    mesh = plsc.VectorSubcoreMesh(core_axis_name="c", subcore_axis_name="s")
    f = pl.kernel(
        _body,
        out_type=jax.ShapeDtypeStruct((_B, _DIM), jnp.float32),
        mesh=mesh,
        compiler_params=pltpu.CompilerParams(use_tc_tiling_on_sc=False),
        scratch_types=[
            pltpu.VMEM((_NB, _S), jnp.int32),
            pltpu.VMEM((_NB, _NGS, _G), jnp.int32),
            pltpu.VMEM((_NB, _S, _DIM), jnp.float32),
            pltpu.SemaphoreType.DMA((_NB,)),
            pltpu.SemaphoreType.DMA((_NB,)),
            pltpu.SemaphoreType.DMA((_NB,)),
        ],
    )
    out = f(xf, tab)
    return out.reshape(x.shape[0], x.shape[1], _DIM)
